# baseline (speedup 1.0000x reference)
import functools

import jax
import jax.numpy as jnp
from jax import lax
from jax.experimental import pallas as pl
from jax.experimental.pallas import tpu as pltpu
from jax.experimental.pallas import tpu_sc as plsc

N = 10000
E = 320000
D = 128
HID = 128
OUT = 10
G = 64
L = 5

NC = 2
NS = 16
NW = NC * NS
K = 64
NBUF = 3
FAST_CORE = 0
FCHUNK = 192
SCHUNK = 124
NCHMAX = FCHUNK
EF = NS * FCHUNK * K
ES = NS * SCHUNK * K
EP = EF + ES
RPT = 632
NP = NS * RPT

ROWBLK = 1000
NBLK = N // ROWBLK


@functools.cache
def _get_sc_scatter():
    def body(x_hbm, pidx_hbm, zeros_hbm, out_hbm,
             pidx_v, s0, s1, s2, s3, d0, d1, d2, d3, rb0, rb1, f0, f1,
             agg_sh, g0, g1, ss0, ss1, isem):
        sidx = [s0, s1, s2, s3]
        didx = [d0, d1, d2, d3]
        rowsb = [rb0, rb1]
        fbuf = [f0, f1]
        gsem = [g0, g1]
        ssem = [ss0, ss1]
        c = lax.axis_index("c")
        s = lax.axis_index("s")
        wid = s * NC + c
        nch = jnp.where(c == FAST_CORE, FCHUNK, SCHUNK)
        ngrp = jnp.where(c == FAST_CORE, FCHUNK // 4, SCHUNK // 4)
        icp = pltpu.async_copy(pidx_hbm.at[wid], pidx_v, isem)
        pltpu.sync_copy(zeros_hbm.at[pl.ds(s * RPT, RPT)],
                        agg_sh.at[pl.ds(s * RPT, RPT)])
        icp.wait()
        plsc.subcore_barrier()

        def unpack(ci, t):
            for q in range(K // 16):
                p = pidx_v[ci, pl.ds(16 * q, 16)]
                sidx[t][pl.ds(16 * q, 16)] = lax.bitwise_and(p, 0xFFFF)
                didx[t][pl.ds(16 * q, 16)] = lax.shift_right_logical(p, 16)

        def widen(b):
            def row(r, carry):
                for w in range(D // 32):
                    q = rowsb[b][r, pl.ds(16 * w, 16)]
                    ev = plsc.bitcast(lax.shift_left(q, 16), jnp.float32)
                    od = plsc.bitcast(
                        lax.bitwise_and(q, jnp.int32(-65536)), jnp.float32)
                    fbuf[b][r, pl.ds(32 * w, 16)] = ev
                    fbuf[b][r, pl.ds(32 * w + 16, 16)] = od
                return carry
            lax.fori_loop(0, K, row, 0)

        for t in range(2):
            unpack(t, t)
            pltpu.async_copy(x_hbm.at[sidx[t]], rowsb[t], gsem[t])

        def group(g, carry):
            for t in range(4):
                ci = g * 4 + t
                b = t % 2
                tp = (t + 2) % 4

                pltpu.make_async_copy(x_hbm.at[sidx[t]], rowsb[b],
                                      gsem[b]).wait()

                @pl.when(ci >= 2)
                def _(b=b):
                    pltpu.make_async_copy(
                        fbuf[b], agg_sh.at[didx[b]], ssem[b]).wait()

                widen(b)
                pltpu.async_copy(fbuf[b], agg_sh.at[didx[t]],
                                 ssem[b], add=True)

                @pl.when(ci + 2 < nch)
                def _(b=b, tp=tp, ci=ci):
                    unpack(ci + 2, tp)
                    pltpu.async_copy(x_hbm.at[sidx[tp]], rowsb[b], gsem[b])
            return carry

        lax.fori_loop(0, ngrp, group, 0)
        for b in range(2):
            pltpu.make_async_copy(fbuf[b], agg_sh.at[didx[b]],
                                  ssem[b]).wait()
        plsc.subcore_barrier()
        pltpu.sync_copy(agg_sh.at[pl.ds(s * RPT, RPT)],
                        out_hbm.at[c, pl.ds(s * RPT, RPT)])

    return pl.kernel(
        body,
        out_type=jax.ShapeDtypeStruct((NC, NP, D), jnp.float32),
        mesh=plsc.VectorSubcoreMesh(core_axis_name="c", subcore_axis_name="s",
                                    num_cores=NC, num_subcores=NS),
        compiler_params=pltpu.CompilerParams(needs_layout_passes=False,
                                             use_tc_tiling_on_sc=False),
        scratch_types=(
            [pltpu.VMEM((NCHMAX, K), jnp.int32)]
            + [pltpu.VMEM((K,), jnp.int32) for _ in range(8)]
            + [pltpu.VMEM((K, D // 2), jnp.int32) for _ in range(2)]
            + [pltpu.VMEM((K, D), jnp.float32) for _ in range(2)]
            + [pltpu.VMEM_SHARED((NP, D), jnp.float32)]
            + [pltpu.SemaphoreType.DMA for _ in range(5)]
        ),
    )


def _sc_scatter(hpacked, pidx, zeros_np):
    return _get_sc_scatter()(hpacked, pidx, zeros_np)


def _pack_bf(h):
    hp = h.reshape(N, D // 32, 2, 16).transpose(0, 1, 3, 2).reshape(N, D)
    hb = hp.astype(jnp.bfloat16).reshape(N, D // 2, 2)
    return lax.bitcast_convert_type(hb, jnp.int32)


def _mlp_body(h_ref, agg_ref, w1_ref, b1_ref, g_ref, be_ref, w2_ref, b2_ref,
              out_ref, t_ref, s1_ref, s2_ref):
    p = pl.program_id(0)
    j = pl.program_id(1)

    @pl.when(p == 0)
    def _phase0():
        m = h_ref[...] + agg_ref[0] + agg_ref[1]
        t = jnp.dot(m, w1_ref[...], preferred_element_type=jnp.float32)
        t = t + b1_ref[...]
        t_ref[pl.ds(j * ROWBLK, ROWBLK), :] = t

        @pl.when(j == 0)
        def _():
            s1_ref[...] = jnp.zeros_like(s1_ref)
            s2_ref[...] = jnp.zeros_like(s2_ref)

        s1_ref[...] += jnp.sum(t, axis=0, keepdims=True)
        s2_ref[...] += jnp.sum(t * t, axis=0, keepdims=True)

    @pl.when(p == 1)
    def _phase1():
        mu = s1_ref[...] / N
        var = s2_ref[...] / N - mu * mu
        rstd = lax.rsqrt(var + 1e-5)
        t = t_ref[pl.ds(j * ROWBLK, ROWBLK), :]
        u = (t - mu) * (rstd * g_ref[...]) + be_ref[...]
        u = jnp.maximum(u, 0.0)
        v = jnp.dot(u, w2_ref[...], preferred_element_type=jnp.float32)
        out_ref[...] = jnp.maximum(v + b2_ref[...], 0.0)


def _mlp(h, aggs, w1, b1, gamma, beta, w2, b2):
    return pl.pallas_call(
        _mlp_body,
        grid=(2, NBLK),
        in_specs=[
            pl.BlockSpec((ROWBLK, D), lambda p, j: ((1 - p) * j, 0)),
            pl.BlockSpec((NC, ROWBLK, D), lambda p, j: (0, (1 - p) * j, 0)),
            pl.BlockSpec((D, HID), lambda p, j: (0, 0)),
            pl.BlockSpec((1, HID), lambda p, j: (0, 0)),
            pl.BlockSpec((1, HID), lambda p, j: (0, 0)),
            pl.BlockSpec((1, HID), lambda p, j: (0, 0)),
            pl.BlockSpec((HID, HID), lambda p, j: (0, 0)),
            pl.BlockSpec((1, HID), lambda p, j: (0, 0)),
        ],
        out_specs=pl.BlockSpec((ROWBLK, HID), lambda p, j: (p * j, 0)),
        out_shape=jax.ShapeDtypeStruct((N, HID), jnp.float32),
        scratch_shapes=[
            pltpu.VMEM((N, HID), jnp.float32),
            pltpu.VMEM((1, HID), jnp.float32),
            pltpu.VMEM((1, HID), jnp.float32),
        ],
    )(h, aggs, w1, b1, gamma, beta, w2, b2)


def _pool_body(h_ref, batch_ref, w1_ref, b1_ref, w2_ref, b2_ref,
               out_ref, acc_ref):
    j = pl.program_id(0)

    @pl.when(j == 0)
    def _():
        acc_ref[...] = jnp.zeros_like(acc_ref)

    seg = lax.broadcasted_iota(jnp.int32, (G, ROWBLK), 0)
    onehot = (seg == batch_ref[0]).astype(jnp.float32)
    acc_ref[...] += jnp.dot(onehot, h_ref[...],
                            preferred_element_type=jnp.float32)

    @pl.when(j == NBLK - 1)
    def _():
        z = jnp.dot(acc_ref[...], w1_ref[...],
                    preferred_element_type=jnp.float32) + b1_ref[...]
        z = jnp.maximum(z, 0.0)
        out_ref[...] = jnp.dot(z, w2_ref[...],
                               preferred_element_type=jnp.float32) + b2_ref[...]


def _pool(h, batch3, lin1_w, lin1_b, lin2_w, lin2_b):
    return pl.pallas_call(
        _pool_body,
        grid=(NBLK,),
        in_specs=[
            pl.BlockSpec((ROWBLK, D), lambda j: (j, 0)),
            pl.BlockSpec((1, 1, ROWBLK), lambda j: (j, 0, 0)),
            pl.BlockSpec((HID, HID), lambda j: (0, 0)),
            pl.BlockSpec((1, HID), lambda j: (0, 0)),
            pl.BlockSpec((HID, OUT), lambda j: (0, 0)),
            pl.BlockSpec((1, OUT), lambda j: (0, 0)),
        ],
        out_specs=pl.BlockSpec((G, OUT), lambda j: (0, 0)),
        out_shape=jax.ShapeDtypeStruct((G, OUT), jnp.float32),
        scratch_shapes=[pltpu.VMEM((G, HID), jnp.float32)],
    )(h, batch3, lin1_w, lin1_b, lin2_w, lin2_b)


def kernel(x, edge_index, batch, W1s, b1s, gammas, betas, W2s, b2s,
           lin1_W, lin1_b, lin2_W, lin2_b):
    src = edge_index[0]
    dst = edge_index[1]
    pad = EP - E
    src_p = jnp.concatenate([src, jnp.zeros((pad,), jnp.int32)])
    dst_p = jnp.concatenate([dst, jnp.full((pad,), N, jnp.int32)])
    packed = (dst_p.astype(jnp.uint32) << 16) | src_p.astype(jnp.uint32)
    packed = lax.bitcast_convert_type(packed, jnp.int32)
    dummy = jnp.array(N << 16, jnp.int32)
    part_f = packed[:EF].reshape(NS, FCHUNK, K)
    part_s = packed[EF:].reshape(NS, SCHUNK, K)
    part_s = jnp.pad(part_s, ((0, 0), (0, NCHMAX - SCHUNK), (0, 0)),
                     constant_values=dummy)
    if FAST_CORE == 0:
        pidx = jnp.stack([part_f, part_s], axis=1).reshape(NW, NCHMAX, K)
    else:
        pidx = jnp.stack([part_s, part_f], axis=1).reshape(NW, NCHMAX, K)
    zeros_np = jnp.zeros((NP, D), jnp.float32)
    batch3 = batch.reshape(NBLK, 1, ROWBLK)

    h = x
    for i in range(L):
        aggs = _sc_scatter(_pack_bf(h), pidx, zeros_np)
        h = _mlp(h, aggs, W1s[i], b1s[i].reshape(1, HID),
                 gammas[i].reshape(1, HID), betas[i].reshape(1, HID),
                 W2s[i], b2s[i].reshape(1, HID))
    return _pool(h, batch3, lin1_W, lin1_b.reshape(1, HID),
                 lin2_W, lin2_b.reshape(1, OUT))


# speedup vs baseline: 1.2982x; 1.2982x over previous
import functools

import jax
import jax.numpy as jnp
from jax import lax
from jax.experimental import pallas as pl
from jax.experimental.pallas import tpu as pltpu
from jax.experimental.pallas import tpu_sc as plsc

N = 10000
E = 320000
D = 128
HID = 128
OUT = 10
G = 64
L = 5

NC = 2
NS = 16
NW = NC * NS
K = 64
NBUF = 3
FAST_CORE = 0
FCHUNK = 192
SCHUNK = 123
NCHMAX = FCHUNK
EF = NS * FCHUNK * K
ES = NS * SCHUNK * K
EP = EF + ES
RPT = 632
NP = NS * RPT

ROWBLK = 1000
NBLK = N // ROWBLK


@functools.cache
def _get_sc_scatter():
    def body(x_hbm, pidx_hbm, zeros_hbm, out_hbm,
             pidx_v, s0, s1, s2, d0, d1, d2, r0, r1, r2,
             agg_sh, g0, g1, g2, ss0, ss1, ss2, isem):
        sidx = [s0, s1, s2]
        didx = [d0, d1, d2]
        rows = [r0, r1, r2]
        gsem = [g0, g1, g2]
        ssem = [ss0, ss1, ss2]
        c = lax.axis_index("c")
        s = lax.axis_index("s")
        wid = s * NC + c
        nch = jnp.where(c == FAST_CORE, FCHUNK, SCHUNK)
        ngrp = jnp.where(c == FAST_CORE, FCHUNK // NBUF, SCHUNK // NBUF)
        icp = pltpu.async_copy(pidx_hbm.at[wid], pidx_v, isem)
        pltpu.sync_copy(zeros_hbm.at[pl.ds(s * RPT, RPT)],
                        agg_sh.at[pl.ds(s * RPT, RPT)])
        icp.wait()
        plsc.subcore_barrier()

        def unpack(ci, b):
            for q in range(K // 16):
                p = pidx_v[ci, pl.ds(16 * q, 16)]
                sidx[b][pl.ds(16 * q, 16)] = lax.bitwise_and(p, 0xFFFF)
                didx[b][pl.ds(16 * q, 16)] = lax.shift_right_logical(p, 16)

        for b in range(2):
            unpack(b, b)
            pltpu.async_copy(x_hbm.at[sidx[b]], rows[b], gsem[b])

        def group(g, carry):
            for b in range(NBUF):
                ci = g * NBUF + b
                cp = ci + 2
                bp = (b + 2) % NBUF

                @pl.when(jnp.logical_and(ci >= 1, cp < nch))
                def _(b=b, bp=bp, cp=cp):
                    pltpu.make_async_copy(
                        rows[bp], agg_sh.at[didx[bp]], ssem[bp]).wait()
                    unpack(cp, bp)
                    pltpu.async_copy(x_hbm.at[sidx[bp]], rows[bp], gsem[bp])

                @pl.when(jnp.logical_and(ci == 0, cp < nch))
                def _(b=b, bp=bp, cp=cp):
                    unpack(cp, bp)
                    pltpu.async_copy(x_hbm.at[sidx[bp]], rows[bp], gsem[bp])

                pltpu.make_async_copy(x_hbm.at[sidx[b]], rows[b],
                                      gsem[b]).wait()
                pltpu.async_copy(rows[b], agg_sh.at[didx[b]],
                                 ssem[b], add=True)
            return carry

        lax.fori_loop(0, ngrp, group, 0)
        for b in range(NBUF):
            pltpu.make_async_copy(rows[b], agg_sh.at[didx[b]],
                                  ssem[b]).wait()
        plsc.subcore_barrier()
        pltpu.sync_copy(agg_sh.at[pl.ds(s * RPT, RPT)],
                        out_hbm.at[c, pl.ds(s * RPT, RPT)])

    return pl.kernel(
        body,
        out_type=jax.ShapeDtypeStruct((NC, NP, D), jnp.float32),
        mesh=plsc.VectorSubcoreMesh(core_axis_name="c", subcore_axis_name="s",
                                    num_cores=NC, num_subcores=NS),
        scratch_types=(
            [pltpu.VMEM((NCHMAX, K), jnp.int32)]
            + [pltpu.VMEM((K,), jnp.int32) for _ in range(2 * NBUF)]
            + [pltpu.VMEM((K, D), jnp.float32) for _ in range(NBUF)]
            + [pltpu.VMEM_SHARED((NP, D), jnp.float32)]
            + [pltpu.SemaphoreType.DMA for _ in range(2 * NBUF + 1)]
        ),
    )


def _sc_scatter(h, pidx, zeros_np):
    return _get_sc_scatter()(h, pidx, zeros_np)


def _mlp_body(h_ref, agg_ref, w1_ref, b1_ref, g_ref, be_ref, w2_ref, b2_ref,
              out_ref, t_ref, s1_ref, s2_ref):
    p = pl.program_id(0)
    j = pl.program_id(1)

    @pl.when(p == 0)
    def _phase0():
        m = h_ref[...] + agg_ref[0] + agg_ref[1]
        t = jnp.dot(m, w1_ref[...], preferred_element_type=jnp.float32)
        t = t + b1_ref[...]
        t_ref[pl.ds(j * ROWBLK, ROWBLK), :] = t

        @pl.when(j == 0)
        def _():
            s1_ref[...] = jnp.zeros_like(s1_ref)
            s2_ref[...] = jnp.zeros_like(s2_ref)

        s1_ref[...] += jnp.sum(t, axis=0, keepdims=True)
        s2_ref[...] += jnp.sum(t * t, axis=0, keepdims=True)

    @pl.when(p == 1)
    def _phase1():
        mu = s1_ref[...] / N
        var = s2_ref[...] / N - mu * mu
        rstd = lax.rsqrt(var + 1e-5)
        t = t_ref[pl.ds(j * ROWBLK, ROWBLK), :]
        u = (t - mu) * (rstd * g_ref[...]) + be_ref[...]
        u = jnp.maximum(u, 0.0)
        v = jnp.dot(u, w2_ref[...], preferred_element_type=jnp.float32)
        out_ref[...] = jnp.maximum(v + b2_ref[...], 0.0)


def _mlp(h, aggs, w1, b1, gamma, beta, w2, b2):
    return pl.pallas_call(
        _mlp_body,
        grid=(2, NBLK),
        in_specs=[
            pl.BlockSpec((ROWBLK, D), lambda p, j: ((1 - p) * j, 0)),
            pl.BlockSpec((NC, ROWBLK, D), lambda p, j: (0, (1 - p) * j, 0)),
            pl.BlockSpec((D, HID), lambda p, j: (0, 0)),
            pl.BlockSpec((1, HID), lambda p, j: (0, 0)),
            pl.BlockSpec((1, HID), lambda p, j: (0, 0)),
            pl.BlockSpec((1, HID), lambda p, j: (0, 0)),
            pl.BlockSpec((HID, HID), lambda p, j: (0, 0)),
            pl.BlockSpec((1, HID), lambda p, j: (0, 0)),
        ],
        out_specs=pl.BlockSpec((ROWBLK, HID), lambda p, j: (p * j, 0)),
        out_shape=jax.ShapeDtypeStruct((N, HID), jnp.float32),
        scratch_shapes=[
            pltpu.VMEM((N, HID), jnp.float32),
            pltpu.VMEM((1, HID), jnp.float32),
            pltpu.VMEM((1, HID), jnp.float32),
        ],
    )(h, aggs, w1, b1, gamma, beta, w2, b2)


def _pool_body(h_ref, batch_ref, w1_ref, b1_ref, w2_ref, b2_ref,
               out_ref, acc_ref):
    j = pl.program_id(0)

    @pl.when(j == 0)
    def _():
        acc_ref[...] = jnp.zeros_like(acc_ref)

    seg = lax.broadcasted_iota(jnp.int32, (G, ROWBLK), 0)
    onehot = (seg == batch_ref[0]).astype(jnp.float32)
    acc_ref[...] += jnp.dot(onehot, h_ref[...],
                            preferred_element_type=jnp.float32)

    @pl.when(j == NBLK - 1)
    def _():
        z = jnp.dot(acc_ref[...], w1_ref[...],
                    preferred_element_type=jnp.float32) + b1_ref[...]
        z = jnp.maximum(z, 0.0)
        out_ref[...] = jnp.dot(z, w2_ref[...],
                               preferred_element_type=jnp.float32) + b2_ref[...]


def _pool(h, batch3, lin1_w, lin1_b, lin2_w, lin2_b):
    return pl.pallas_call(
        _pool_body,
        grid=(NBLK,),
        in_specs=[
            pl.BlockSpec((ROWBLK, D), lambda j: (j, 0)),
            pl.BlockSpec((1, 1, ROWBLK), lambda j: (j, 0, 0)),
            pl.BlockSpec((HID, HID), lambda j: (0, 0)),
            pl.BlockSpec((1, HID), lambda j: (0, 0)),
            pl.BlockSpec((HID, OUT), lambda j: (0, 0)),
            pl.BlockSpec((1, OUT), lambda j: (0, 0)),
        ],
        out_specs=pl.BlockSpec((G, OUT), lambda j: (0, 0)),
        out_shape=jax.ShapeDtypeStruct((G, OUT), jnp.float32),
        scratch_shapes=[pltpu.VMEM((G, HID), jnp.float32)],
    )(h, batch3, lin1_w, lin1_b, lin2_w, lin2_b)


def kernel(x, edge_index, batch, W1s, b1s, gammas, betas, W2s, b2s,
           lin1_W, lin1_b, lin2_W, lin2_b):
    src = edge_index[0]
    dst = edge_index[1]
    pad = EP - E
    src_p = jnp.concatenate([src, jnp.zeros((pad,), jnp.int32)])
    dst_p = jnp.concatenate([dst, jnp.full((pad,), N, jnp.int32)])
    packed = (dst_p.astype(jnp.uint32) << 16) | src_p.astype(jnp.uint32)
    packed = lax.bitcast_convert_type(packed, jnp.int32)
    dummy = jnp.array(N << 16, jnp.int32)
    part_f = packed[:EF].reshape(NS, FCHUNK, K)
    part_s = packed[EF:].reshape(NS, SCHUNK, K)
    part_s = jnp.pad(part_s, ((0, 0), (0, NCHMAX - SCHUNK), (0, 0)),
                     constant_values=dummy)
    if FAST_CORE == 0:
        pidx = jnp.stack([part_f, part_s], axis=1).reshape(NW, NCHMAX, K)
    else:
        pidx = jnp.stack([part_s, part_f], axis=1).reshape(NW, NCHMAX, K)
    zeros_np = jnp.zeros((NP, D), jnp.float32)
    batch3 = batch.reshape(NBLK, 1, ROWBLK)

    h = x
    for i in range(L):
        aggs = _sc_scatter(h, pidx, zeros_np)
        h = _mlp(h, aggs, W1s[i], b1s[i].reshape(1, HID),
                 gammas[i].reshape(1, HID), betas[i].reshape(1, HID),
                 W2s[i], b2s[i].reshape(1, HID))
    return _pool(h, batch3, lin1_W, lin1_b.reshape(1, HID),
                 lin2_W, lin2_b.reshape(1, OUT))
